# Initial kernel scaffold; baseline (speedup 1.0000x reference)
import functools

import jax
import jax.numpy as jnp
from jax import lax
from jax.experimental import pallas as pl
from jax.experimental.pallas import tpu as pltpu
from jax.experimental.pallas import tpu_sc as plsc

N_ELEMS = 16384 * 200
NW = 32
PER_W = N_ELEMS // NW
LANES = 16


def _sc_kernel(x_hbm, t_hbm, out_hbm, xbuf, tbuf, accbuf):
    wid = lax.axis_index("s") * 2 + lax.axis_index("c")
    base = wid * PER_W

    pltpu.sync_copy(x_hbm.at[pl.ds(base, PER_W)], xbuf)
    pltpu.sync_copy(t_hbm, tbuf)

    rowsum = tbuf[0, :] + tbuf[1, :] + tbuf[2, :]
    accbuf[...] = rowsum
    lookup = accbuf

    def body(i, acc):
        idx = xbuf[pl.ds(i * LANES, LANES)]
        return acc + plsc.load_gather(lookup, [idx])

    acc = lax.fori_loop(0, PER_W // LANES, body,
                        jnp.zeros((LANES,), jnp.float32))
    accbuf[...] = acc
    pltpu.sync_copy(accbuf, out_hbm.at[wid])


@jax.jit
def kernel(x, table):
    x_flat = x.reshape(-1)
    t_pad = jnp.zeros((3, LANES), jnp.float32).at[:, :10].set(table.T)

    k = functools.partial(
        pl.kernel,
        mesh=plsc.VectorSubcoreMesh(core_axis_name="c", subcore_axis_name="s"),
        out_type=jax.ShapeDtypeStruct((NW, LANES), jnp.float32),
        scratch_types=[
            pltpu.VMEM((PER_W,), jnp.int32),
            pltpu.VMEM((3, LANES), jnp.float32),
            pltpu.VMEM((LANES,), jnp.float32),
        ],
    )(_sc_kernel)
    partials = k(x_flat, t_pad)
    return partials.sum()


# speedup vs baseline: 122.3341x; 122.3341x over previous
import functools

import jax
import jax.numpy as jnp
from jax import lax
from jax.experimental import pallas as pl
from jax.experimental.pallas import tpu as pltpu
from jax.experimental.pallas import tpu_sc as plsc

N_ELEMS = 16384 * 200
NW = 32
PER_W = N_ELEMS // NW
LANES = 16


def _sc_kernel(x_hbm, t_hbm, out_hbm, xbuf, tbuf, accbuf):
    wid = lax.axis_index("s") * 2 + lax.axis_index("c")
    base = wid * PER_W

    pltpu.sync_copy(x_hbm.at[pl.ds(base, PER_W)], xbuf)
    pltpu.sync_copy(t_hbm, tbuf)

    rowsum = tbuf[0, :] + tbuf[1, :] + tbuf[2, :]
    accbuf[...] = rowsum
    lookup = accbuf

    def body(i, acc):
        idx = xbuf[pl.ds(i * LANES, LANES)]
        return acc + plsc.load_gather(lookup, [idx])

    acc = lax.fori_loop(0, PER_W // LANES, body,
                        jnp.zeros((LANES,), jnp.float32))
    accbuf[...] = acc
    pltpu.sync_copy(accbuf, out_hbm.at[wid])


@jax.jit
def kernel(x, table):
    x_flat = x.reshape(-1)
    t_pad = jnp.zeros((3, LANES), jnp.float32).at[:, :10].set(table.T)

    k = functools.partial(
        pl.kernel,
        mesh=plsc.VectorSubcoreMesh(core_axis_name="c", subcore_axis_name="s"),
        out_type=jax.ShapeDtypeStruct((NW, LANES), jnp.float32),
        compiler_params=pltpu.CompilerParams(needs_layout_passes=False),
        scratch_types=[
            pltpu.VMEM((PER_W,), jnp.int32),
            pltpu.VMEM((3, LANES), jnp.float32),
            pltpu.VMEM((LANES,), jnp.float32),
        ],
    )(_sc_kernel)
    partials = k(x_flat, t_pad)
    return partials.sum()


# speedup vs baseline: 177.5136x; 1.4511x over previous
import functools

import jax
import jax.numpy as jnp
from jax import lax
from jax.experimental import pallas as pl
from jax.experimental.pallas import tpu as pltpu
from jax.experimental.pallas import tpu_sc as plsc

N_ELEMS = 16384 * 200
NW = 32
PER_W = N_ELEMS // NW
LANES = 16


def _sc_kernel(x_hbm, t_hbm, out_hbm, xbuf, tbuf, accbuf):
    wid = lax.axis_index("s") * 2 + lax.axis_index("c")
    base = wid * PER_W

    pltpu.sync_copy(x_hbm.at[pl.ds(base, PER_W)], xbuf)
    pltpu.sync_copy(t_hbm, tbuf)

    rowsum = tbuf[0, :] + tbuf[1, :] + tbuf[2, :]
    accbuf[...] = rowsum
    lookup = accbuf

    UNROLL = 16

    def body(i, accs):
        base_i = i * (LANES * UNROLL)
        accs = list(accs)
        for u in range(UNROLL):
            idx = xbuf[pl.ds(base_i + u * LANES, LANES)]
            accs[u % 4] = accs[u % 4] + plsc.load_gather(lookup, [idx])
        return tuple(accs)

    zero = jnp.zeros((LANES,), jnp.float32)
    a0, a1, a2, a3 = lax.fori_loop(0, PER_W // (LANES * UNROLL), body,
                                   (zero, zero, zero, zero))
    accbuf[...] = (a0 + a1) + (a2 + a3)
    pltpu.sync_copy(accbuf, out_hbm.at[wid])


@jax.jit
def kernel(x, table):
    x_flat = x.reshape(-1)
    t_pad = jnp.zeros((3, LANES), jnp.float32).at[:, :10].set(table.T)

    k = functools.partial(
        pl.kernel,
        mesh=plsc.VectorSubcoreMesh(core_axis_name="c", subcore_axis_name="s"),
        out_type=jax.ShapeDtypeStruct((NW, LANES), jnp.float32),
        compiler_params=pltpu.CompilerParams(needs_layout_passes=False),
        scratch_types=[
            pltpu.VMEM((PER_W,), jnp.int32),
            pltpu.VMEM((3, LANES), jnp.float32),
            pltpu.VMEM((LANES,), jnp.float32),
        ],
    )(_sc_kernel)
    partials = k(x_flat, t_pad)
    return partials.sum()


# speedup vs baseline: 272.1461x; 1.5331x over previous
import functools

import jax
import jax.numpy as jnp
from jax import lax
from jax.experimental import pallas as pl
from jax.experimental.pallas import tpu as pltpu
from jax.experimental.pallas import tpu_sc as plsc

ROWS = 16384
COLS = 200
NW = 32
ROWS_W = ROWS // NW
CHUNK_R = 128
NCHUNK = ROWS_W // CHUNK_R
LANES = 16
ROWS_PER_IT = 2


def _sc_kernel(x_hbm, t_hbm, out_hbm, xb0, xb1, tbuf, accbuf, sem0, sem1):
    wid = lax.axis_index("s") * 2 + lax.axis_index("c")
    base = wid * ROWS_W

    pltpu.sync_copy(t_hbm, tbuf)
    rowsum = tbuf[0, :] + tbuf[1, :] + tbuf[2, :]
    accbuf[...] = rowsum
    lookup = accbuf

    lane = lax.iota(jnp.int32, 16)
    head8 = lane < 8

    bufs = (xb0, xb1)
    sems = (sem0, sem1)

    def chunk_body(xb):
        def body(i, accs):
            accs = list(accs)
            for rr in range(ROWS_PER_IT):
                r = i * ROWS_PER_IT + rr
                for k in range(12):
                    idx = xb[r, pl.ds(k * LANES, LANES)]
                    g = plsc.load_gather(lookup, [idx])
                    accs[k % 4] = accs[k % 4] + g
                tail = xb[r, pl.ds(184, LANES)]
                tail = jnp.where(head8, 10, tail)
                g = plsc.load_gather(lookup, [tail])
                accs[rr] = accs[rr] + g
            return tuple(accs)
        return body

    zero = jnp.zeros((LANES,), jnp.float32)
    accs = (zero, zero, zero, zero)

    copies = [None] * NCHUNK
    copies[0] = pltpu.async_copy(
        x_hbm.at[pl.ds(base, CHUNK_R)], bufs[0], sems[0])
    for c in range(NCHUNK):
        copies[c].wait()
        if c + 1 < NCHUNK:
            copies[c + 1] = pltpu.async_copy(
                x_hbm.at[pl.ds(base + (c + 1) * CHUNK_R, CHUNK_R)],
                bufs[(c + 1) % 2], sems[(c + 1) % 2])
        accs = lax.fori_loop(0, CHUNK_R // ROWS_PER_IT,
                             chunk_body(bufs[c % 2]), accs)

    accbuf[...] = (accs[0] + accs[1]) + (accs[2] + accs[3])
    pltpu.sync_copy(accbuf, out_hbm.at[wid])


@jax.jit
def kernel(x, table):
    t_pad = jnp.zeros((3, LANES), jnp.float32).at[:, :10].set(table.T)

    k = functools.partial(
        pl.kernel,
        mesh=plsc.VectorSubcoreMesh(core_axis_name="c", subcore_axis_name="s"),
        out_type=jax.ShapeDtypeStruct((NW, LANES), jnp.float32),
        compiler_params=pltpu.CompilerParams(needs_layout_passes=False),
        scratch_types=[
            pltpu.VMEM((CHUNK_R, COLS), jnp.int32),
            pltpu.VMEM((CHUNK_R, COLS), jnp.int32),
            pltpu.VMEM((3, LANES), jnp.float32),
            pltpu.VMEM((LANES,), jnp.float32),
            pltpu.SemaphoreType.DMA,
            pltpu.SemaphoreType.DMA,
        ],
    )(_sc_kernel)
    partials = k(x, t_pad)
    return partials.sum()
